# baseline (speedup 1.0000x reference)
import functools

import jax
import jax.numpy as jnp
from jax import lax
from jax.experimental import pallas as pl
from jax.experimental.pallas import tpu as pltpu
from jax.experimental.pallas import tpu_sc as plsc

BATCH = 32
SEQ_LEN = 8192
D = 128
NUM_EMB = 6
NUM_PAIR = NUM_EMB * NUM_EMB
D2 = 2 * D
TOTAL = BATCH * SEQ_LEN
PAIRS = TOTAL // 2
NUM_CORES = 2
NUM_SUBCORES = 16
NW = NUM_CORES * NUM_SUBCORES
IPW = TOTAL // NW
PPW = PAIRS // NW
CH = 128
NCH = PPW // CH
NBUF = 2

_mesh = plsc.VectorSubcoreMesh(core_axis_name="c", subcore_axis_name="s")


@functools.partial(
    pl.kernel,
    mesh=_mesh,
    out_type=jax.ShapeDtypeStruct((PAIRS, 2, D), jnp.float32),
    scratch_types=[
        pltpu.VMEM((IPW,), jnp.int32),
        pltpu.VMEM((NCH, CH), jnp.int32),
        pltpu.VMEM_SHARED((NUM_PAIR, 2, D), jnp.float32),
        pltpu.VMEM((NBUF, CH, 2, D), jnp.float32),
        pltpu.SemaphoreType.DMA,
        pltpu.SemaphoreType.DMA,
        pltpu.SemaphoreType.DMA,
        pltpu.SemaphoreType.DMA,
    ],
)
def _emb_lookup(x_hbm, table_hbm, out_hbm, idx_v, cidx_v, ptab_v, rows_v,
                gsem0, gsem1, wsem0, wsem1):
    gsem = (gsem0, gsem1)
    wsem = (wsem0, wsem1)
    sid = lax.axis_index("s")
    wid = sid * NUM_CORES + lax.axis_index("c")
    base = wid * PPW

    for k in range(NUM_PAIR):
        @pl.when(sid == k % NUM_SUBCORES)
        def _(k=k):
            pltpu.sync_copy(table_hbm.at[k // NUM_EMB], ptab_v.at[k, 0])
            pltpu.sync_copy(table_hbm.at[k % NUM_EMB], ptab_v.at[k, 1])

    pltpu.sync_copy(x_hbm.at[pl.ds(wid * IPW, IPW)], idx_v)

    iota = lax.iota(jnp.int32, 16)
    ev = (iota * 2) & 15
    od = (iota * 2 + 1) & 15
    low = iota < 8

    def combine(jj, _):
        for g in range(CH // 16):
            o = jj * (2 * CH) + g * 32
            a = idx_v[pl.ds(o, 16)]
            b = idx_v[pl.ds(o + 16, 16)]
            ca = jnp.take(a, ev) * NUM_EMB + \
                jnp.take(a, od)
            cb = jnp.take(b, ev) * NUM_EMB + \
                jnp.take(b, od)
            cidx_v[jj, pl.ds(g * 16, 16)] = jnp.where(low, ca, cb)
        return ()

    lax.fori_loop(0, NCH, combine, (), unroll=False)
    plsc.subcore_barrier()

    for b in range(NBUF):
        pltpu.async_copy(ptab_v.at[cidx_v.at[b]], rows_v.at[b], gsem[b])

    def body(j, _):
        for b in range(NBUF):
            jj = j + b
            pltpu.make_async_copy(ptab_v.at[cidx_v.at[jj]], rows_v.at[b],
                                  gsem[b]).wait()
            pltpu.async_copy(rows_v.at[b],
                             out_hbm.at[pl.ds(base + jj * CH, CH)], wsem[b])
        for b in range(NBUF):
            jj = j + b
            pltpu.make_async_copy(
                rows_v.at[b], out_hbm.at[pl.ds(base + jj * CH, CH)],
                wsem[b]).wait()

            @pl.when(jj + NBUF < NCH)
            def _(jj=jj, b=b):
                pltpu.async_copy(ptab_v.at[cidx_v.at[jj + NBUF]],
                                 rows_v.at[b], gsem[b])
        return ()

    lax.fori_loop(0, NCH // NBUF, lambda i, c: body(i * NBUF, c), (),
                  unroll=False)


def kernel(x, table):
    x1 = x.reshape(TOTAL).astype(jnp.int32)
    out = _emb_lookup(x1, table)
    return out.reshape(BATCH, SEQ_LEN, D)


# speedup vs baseline: 1.3733x; 1.3733x over previous
import functools

import jax
import jax.numpy as jnp
from jax import lax
from jax.experimental import pallas as pl
from jax.experimental.pallas import tpu as pltpu
from jax.experimental.pallas import tpu_sc as plsc

BATCH = 32
SEQ_LEN = 8192
D = 128
NUM_EMB = 6
TOTAL = BATCH * SEQ_LEN
NUM_CORES = 2
NUM_SUBCORES = 16
NW = NUM_CORES * NUM_SUBCORES
BPW = TOTAL // NW
CH = 128
CHH = CH // 2
NCH = BPW // CH
NBUF = 2

_mesh = plsc.VectorSubcoreMesh(core_axis_name="c", subcore_axis_name="s")


@functools.partial(
    pl.kernel,
    mesh=_mesh,
    out_type=jax.ShapeDtypeStruct((TOTAL, D), jnp.float32),
    scratch_types=[
        pltpu.VMEM((NCH, CH), jnp.int32),
        pltpu.VMEM_SHARED((NUM_EMB, D), jnp.float32),
        pltpu.VMEM((NBUF, CH, D), jnp.float32),
        pltpu.SemaphoreType.DMA,
        pltpu.SemaphoreType.DMA,
        pltpu.SemaphoreType.DMA,
        pltpu.SemaphoreType.DMA,
        pltpu.SemaphoreType.DMA,
        pltpu.SemaphoreType.DMA,
    ],
)
def _emb_lookup(x_hbm, table_hbm, out_hbm, idx_v, tab_v, rows_v,
                g0a, g0b, g1a, g1b, wsem0, wsem1):
    gsem = ((g0a, g0b), (g1a, g1b))
    wsem = (wsem0, wsem1)
    wid = lax.axis_index("s") * NUM_CORES + lax.axis_index("c")
    base = wid * BPW

    @pl.when(lax.axis_index("s") == 0)
    def _():
        pltpu.sync_copy(table_hbm, tab_v)

    pltpu.sync_copy(x_hbm.at[pl.ds(wid * NCH, NCH)], idx_v)
    plsc.subcore_barrier()

    def start_gathers(jj, b):
        for h in range(2):
            pltpu.async_copy(tab_v.at[idx_v.at[jj, pl.ds(h * CHH, CHH)]],
                             rows_v.at[b, pl.ds(h * CHH, CHH)], gsem[b][h])

    def wait_gathers(jj, b):
        for h in range(2):
            pltpu.make_async_copy(
                tab_v.at[idx_v.at[jj, pl.ds(h * CHH, CHH)]],
                rows_v.at[b, pl.ds(h * CHH, CHH)], gsem[b][h]).wait()

    for b in range(NBUF):
        start_gathers(b, b)

    def body(j, _):
        for b in range(NBUF):
            jj = j + b
            wait_gathers(jj, b)
            pltpu.async_copy(rows_v.at[b],
                             out_hbm.at[pl.ds(base + jj * CH, CH)], wsem[b])
        for b in range(NBUF):
            jj = j + b
            pltpu.make_async_copy(
                rows_v.at[b], out_hbm.at[pl.ds(base + jj * CH, CH)],
                wsem[b]).wait()

            @pl.when(jj + NBUF < NCH)
            def _(jj=jj, b=b):
                start_gathers(jj + NBUF, b)
        return ()

    lax.fori_loop(0, NCH // NBUF, lambda i, c: body(i * NBUF, c), (),
                  unroll=False)


def kernel(x, table):
    x2 = x.reshape(TOTAL // CH, CH).astype(jnp.int32)
    out = _emb_lookup(x2, table)
    return out.reshape(BATCH, SEQ_LEN, D)
